# Initial kernel scaffold; baseline (speedup 1.0000x reference)
import functools

import jax
import jax.numpy as jnp
from jax import lax
from jax.experimental import pallas as pl
from jax.experimental.pallas import tpu as pltpu
from jax.experimental.pallas import tpu_sc as plsc

D = 64
NC, NS = 2, 16
NW = NC * NS
GROUP = 128
CHUNK = 512


@functools.partial(jax.jit, static_argnums=(2,))
def _gather(idx2d, table, rows_total):
    b_per_w = rows_total // NW
    n_chunks = b_per_w // CHUNK
    groups_per_chunk = CHUNK // GROUP

    mesh = plsc.VectorSubcoreMesh(core_axis_name="c", subcore_axis_name="s")

    @functools.partial(
        pl.kernel,
        mesh=mesh,
        out_type=jax.ShapeDtypeStruct((rows_total, D), jnp.float32),
        scratch_types=[
            pltpu.VMEM((groups_per_chunk, GROUP), jnp.int32),
            pltpu.VMEM((CHUNK, D), jnp.float32),
            pltpu.SemaphoreType.DMA,
        ],
    )
    def k(idx_hbm, table_hbm, out_hbm, idx_v, rows_v, sem):
        wid = lax.axis_index("s") * NC + lax.axis_index("c")
        idx_row_base = wid * (b_per_w // GROUP)
        row_base = wid * b_per_w

        def body(ci, _):
            pltpu.sync_copy(
                idx_hbm.at[pl.ds(idx_row_base + ci * groups_per_chunk,
                                 groups_per_chunk)],
                idx_v,
            )
            descs = []
            for g in range(groups_per_chunk):
                descs.append(pltpu.async_copy(
                    table_hbm.at[idx_v.at[g]],
                    rows_v.at[pl.ds(g * GROUP, GROUP)],
                    sem,
                ))
            for d in descs:
                d.wait()
            pltpu.sync_copy(
                rows_v,
                out_hbm.at[pl.ds(row_base + ci * CHUNK, CHUNK)],
            )
            return 0

        lax.fori_loop(0, n_chunks, body, 0)

    return k(idx2d, table)


def kernel(inputs, time_encodings):
    rows_total = inputs.shape[0] * inputs.shape[1]
    idx2d = inputs.reshape(rows_total // GROUP, GROUP).astype(jnp.int32)
    out = _gather(idx2d, time_encodings, rows_total)
    return out.reshape(inputs.shape[0], inputs.shape[1], D)


# speedup vs baseline: 4.7557x; 4.7557x over previous
import functools

import jax
import jax.numpy as jnp
from jax import lax
from jax.experimental import pallas as pl
from jax.experimental.pallas import tpu as pltpu
from jax.experimental.pallas import tpu_sc as plsc

D = 64
NC, NS = 2, 16
NW = NC * NS
GROUP = 128
CHUNK = 512


@functools.partial(jax.jit, static_argnums=(2,))
def _gather(idx2d, table, rows_total):
    b_per_w = rows_total // NW
    n_chunks = b_per_w // CHUNK
    groups_per_chunk = CHUNK // GROUP

    mesh = plsc.VectorSubcoreMesh(core_axis_name="c", subcore_axis_name="s")

    @functools.partial(
        pl.kernel,
        mesh=mesh,
        out_type=jax.ShapeDtypeStruct((rows_total, D), jnp.float32),
        scratch_types=[
            pltpu.VMEM((groups_per_chunk, GROUP), jnp.int32),
            pltpu.VMEM((CHUNK, D), jnp.float32),
            pltpu.SemaphoreType.DMA,
        ],
        compiler_params=pltpu.CompilerParams(use_tc_tiling_on_sc=False),
    )
    def k(idx_hbm, table_hbm, out_hbm, idx_v, rows_v, sem):
        wid = lax.axis_index("s") * NC + lax.axis_index("c")
        idx_row_base = wid * (b_per_w // GROUP)
        row_base = wid * b_per_w

        def body(ci, _):
            pltpu.sync_copy(
                idx_hbm.at[pl.ds(idx_row_base + ci * groups_per_chunk,
                                 groups_per_chunk)],
                idx_v,
            )
            descs = []
            for g in range(groups_per_chunk):
                descs.append(pltpu.async_copy(
                    table_hbm.at[idx_v.at[g]],
                    rows_v.at[pl.ds(g * GROUP, GROUP)],
                    sem,
                ))
            for d in descs:
                d.wait()
            pltpu.sync_copy(
                rows_v,
                out_hbm.at[pl.ds(row_base + ci * CHUNK, CHUNK)],
            )
            return 0

        lax.fori_loop(0, n_chunks, body, 0)

    return k(idx2d, table)


def kernel(inputs, time_encodings):
    rows_total = inputs.shape[0] * inputs.shape[1]
    idx2d = inputs.reshape(rows_total // GROUP, GROUP).astype(jnp.int32)
    out = _gather(idx2d, time_encodings, rows_total)
    return out.reshape(inputs.shape[0], inputs.shape[1], D)


# speedup vs baseline: 4.9150x; 1.0335x over previous
import functools

import jax
import jax.numpy as jnp
from jax import lax
from jax.experimental import pallas as pl
from jax.experimental.pallas import tpu as pltpu
from jax.experimental.pallas import tpu_sc as plsc

D = 64
NC, NS = 2, 16
NW = NC * NS
NB = 4


def _gather(idx, table):
    B, T = idx.shape
    b_per_w = B // NW
    n_chunks = b_per_w // NB
    seg_bounds = []
    off = 0
    while off < T:
        seg_bounds.append((off, min(128, T - off)))
        off += 128

    mesh = plsc.VectorSubcoreMesh(core_axis_name="c", subcore_axis_name="s")

    @functools.partial(
        pl.kernel,
        mesh=mesh,
        out_type=jax.ShapeDtypeStruct((B, T, D), jnp.float32),
        scratch_types=[
            pltpu.VMEM((NB, T), jnp.int32),
            pltpu.VMEM((NB, T, D), jnp.float32),
            pltpu.SemaphoreType.DMA,
        ],
        compiler_params=pltpu.CompilerParams(use_tc_tiling_on_sc=False),
    )
    def k(idx_hbm, table_hbm, out_hbm, idx_v, rows_v, sem):
        wid = lax.axis_index("s") * NC + lax.axis_index("c")
        b_base = wid * b_per_w

        def body(ci, _):
            b0 = b_base + ci * NB
            pltpu.sync_copy(idx_hbm.at[pl.ds(b0, NB)], idx_v)
            descs = []
            for r in range(NB):
                for (s0, sl) in seg_bounds:
                    descs.append(pltpu.async_copy(
                        table_hbm.at[idx_v.at[r, pl.ds(s0, sl)]],
                        rows_v.at[r, pl.ds(s0, sl)],
                        sem,
                    ))
            for dsc in descs:
                dsc.wait()
            pltpu.sync_copy(rows_v, out_hbm.at[pl.ds(b0, NB)])
            return 0

        lax.fori_loop(0, n_chunks, body, 0)

    return k(idx, table)


def kernel(inputs, time_encodings):
    return _gather(inputs.astype(jnp.int32), time_encodings)
